# baseline (speedup 1.0000x reference)
import functools

import jax
import jax.numpy as jnp
from jax import lax
from jax.experimental import pallas as pl
from jax.experimental.pallas import tpu as pltpu
from jax.experimental.pallas import tpu_sc as plsc

D_MODEL = 768
D_VOCAB = 100000
NC = 2
NS = 16
NW = NC * NS
T = 8192
TPW = T // NW
GT = 16
NG = TPW // GT
IDX_PER_G = GT * D_MODEL


def _embed_body(w_hbm, x_hbm, out_hbm, xv, idx0, idx1, g0, g1,
                gs0, gs1, os0, os1):
    wid = lax.axis_index("s") * NC + lax.axis_index("c")
    tok0 = wid * TPW
    pltpu.sync_copy(x_hbm.at[pl.ds(tok0, TPW)], xv)

    iota = lax.iota(jnp.int32, 16)
    pos_base = iota * D_MODEL
    idxs = (idx0, idx1)
    gbufs = (g0, g1)
    gsems = (gs0, gs1)
    osems = (os0, os1)
    gather_cp = [None, None]
    out_cp = [None, None]

    def build(b, g):
        v_vec = xv[pl.ds(g * GT, GT)]
        ref = idxs[b]

        def body(d, c):
            pos = pos_base + d
            val = v_vec + d * D_VOCAB
            plsc.store_scatter(ref, [pos], val)
            return c

        lax.fori_loop(0, D_MODEL, body, 0)

    def fire_out(b, g):
        w0 = (tok0 + g * GT) * D_MODEL
        return pltpu.async_copy(gbufs[b], out_hbm.at[pl.ds(w0, IDX_PER_G)],
                                osems[b])

    for g in range(NG):
        b = g & 1
        build(b, g)
        if out_cp[b] is not None:
            out_cp[b].wait()
        gather_cp[b] = pltpu.async_copy(w_hbm.at[idxs[b]], gbufs[b],
                                        gsems[b])
        if g >= 1:
            pb = (g - 1) & 1
            gather_cp[pb].wait()
            out_cp[pb] = fire_out(pb, g - 1)

    b = (NG - 1) & 1
    gather_cp[b].wait()
    out_cp[b] = fire_out(b, NG - 1)
    out_cp[0].wait()
    out_cp[1].wait()


@functools.partial(
    pl.kernel,
    out_type=jax.ShapeDtypeStruct((T * D_MODEL,), jnp.float32),
    mesh=plsc.VectorSubcoreMesh(core_axis_name="c", subcore_axis_name="s"),
    compiler_params=pltpu.CompilerParams(needs_layout_passes=False),
    scratch_types=[
        pltpu.VMEM((TPW,), jnp.int32),
        pltpu.VMEM((IDX_PER_G,), jnp.int32),
        pltpu.VMEM((IDX_PER_G,), jnp.int32),
        pltpu.VMEM((IDX_PER_G,), jnp.float32),
        pltpu.VMEM((IDX_PER_G,), jnp.float32),
        pltpu.SemaphoreType.DMA,
        pltpu.SemaphoreType.DMA,
        pltpu.SemaphoreType.DMA,
        pltpu.SemaphoreType.DMA,
    ],
)
def _embed_call(w_hbm, x_hbm, out_hbm, xv, idx0, idx1, g0, g1,
                gs0, gs1, os0, os1):
    _embed_body(w_hbm, x_hbm, out_hbm, xv, idx0, idx1, g0, g1,
                gs0, gs1, os0, os1)


def kernel(x, W_E):
    b, s = x.shape
    xf = x.reshape(-1).astype(jnp.int32)
    wf = W_E.reshape(-1)
    out = _embed_call(wf, xf)
    return out.reshape(b, s, D_MODEL)
